# Initial kernel scaffold; baseline (speedup 1.0000x reference)
import jax
import jax.numpy as jnp
from jax.experimental import pallas as pl


def kernel(x, edge_index, W1, b1, W2, b2):
    raise NotImplementedError("write your pallas kernel here")


# speedup vs baseline: 21.6604x; 21.6604x over previous
import functools

import jax
import jax.numpy as jnp
from jax import lax
from jax.experimental import pallas as pl
from jax.experimental.pallas import tpu as pltpu
from jax.experimental.pallas import tpu_sc as plsc

NC = 2
NS = 16
NW = NC * NS


def _make_mesh():
    return plsc.VectorSubcoreMesh(core_axis_name="c", subcore_axis_name="s")


def _make_deg(npad, rt, ch1):
    @functools.partial(
        pl.kernel,
        mesh=_make_mesh(),
        out_type=jax.ShapeDtypeStruct((npad,), jnp.float32),
        scratch_types=[
            pltpu.VMEM((ch1, 128), jnp.int32),
            pltpu.VMEM((128,), jnp.float32),
            pltpu.VMEM_SHARED((npad,), jnp.float32),
        ],
    )
    def kfn(dstp_hbm, zcol_hbm, deg_hbm, dst_i, ones_v, deg_sh):
        cid = lax.axis_index("c")
        sid = lax.axis_index("s")

        @pl.when(cid == 0)
        def _():
            pltpu.sync_copy(zcol_hbm, deg_sh.at[pl.ds(sid * rt, rt)])

            def fill(k, carry):
                ones_v[pl.ds(k * 16, 16)] = jnp.full((16,), 1.0, jnp.float32)
                return carry

            lax.fori_loop(0, 8, fill, 0)
            pltpu.sync_copy(dstp_hbm.at[pl.ds(sid * ch1, ch1), :], dst_i)
            plsc.subcore_barrier()

            def body(j, carry):
                pltpu.sync_copy(ones_v, deg_sh.at[dst_i.at[j]], add=True)
                return carry

            lax.fori_loop(0, ch1, body, 0)
            plsc.subcore_barrier()
            pltpu.sync_copy(deg_sh.at[pl.ds(sid * rt, rt)],
                            deg_hbm.at[pl.ds(sid * rt, rt)])

    return kfn


def _make_agg(npad, d, rt, chw):
    @functools.partial(
        pl.kernel,
        mesh=_make_mesh(),
        out_type=jax.ShapeDtypeStruct((NC, npad, d), jnp.float32),
        scratch_types=[
            pltpu.VMEM((chw, 128), jnp.int32),
            pltpu.VMEM((chw, 128), jnp.int32),
            pltpu.VMEM((128, d), jnp.float32),
            pltpu.VMEM_SHARED((npad, d), jnp.float32),
            pltpu.SemaphoreType.DMA,
        ],
    )
    def kfn(y_hbm, srcp_hbm, dstp_hbm, zrows_hbm, out_hbm, src_i, dst_i, rows,
            acc, sem):
        cid = lax.axis_index("c")
        sid = lax.axis_index("s")
        wid = cid * NS + sid

        pltpu.sync_copy(zrows_hbm, acc.at[pl.ds(sid * rt, rt), :])
        pltpu.sync_copy(srcp_hbm.at[pl.ds(wid * chw, chw), :], src_i)
        pltpu.sync_copy(dstp_hbm.at[pl.ds(wid * chw, chw), :], dst_i)
        plsc.subcore_barrier()

        def body(j, carry):
            pltpu.async_copy(y_hbm.at[src_i.at[j]], rows, sem).wait()
            pltpu.sync_copy(rows, acc.at[dst_i.at[j]], add=True)
            return carry

        lax.fori_loop(0, chw, body, 0)
        plsc.subcore_barrier()
        pltpu.sync_copy(acc.at[pl.ds(sid * rt, rt), :],
                        out_hbm.at[cid, pl.ds(sid * rt, rt), :])

    return kfn


def _mm_scale(x_pad, w, deg2d, npad, d, br):
    def body(x_ref, w_ref, deg_ref, y_ref):
        dinv = lax.rsqrt(deg_ref[...] + 1.0)
        y_ref[...] = (
            jnp.dot(x_ref[...], w_ref[...], preferred_element_type=jnp.float32)
            * dinv
        )

    return pl.pallas_call(
        body,
        grid=(npad // br,),
        in_specs=[
            pl.BlockSpec((br, d), lambda i: (i, 0)),
            pl.BlockSpec((d, d), lambda i: (0, 0)),
            pl.BlockSpec((br, 1), lambda i: (i, 0)),
        ],
        out_specs=pl.BlockSpec((br, d), lambda i: (i, 0)),
        out_shape=jax.ShapeDtypeStruct((npad, d), jnp.float32),
    )(x_pad, w, deg2d)


def _finish_mm(parts, y, deg2d, b2d, w, npad, d, br):
    def body(p0_ref, p1_ref, y_ref, deg_ref, b_ref, w_ref, h_ref, y2_ref):
        dinv = lax.rsqrt(deg_ref[...] + 1.0)
        h = dinv * (p0_ref[0] + p1_ref[0] + y_ref[...]) + b_ref[...]
        h = jnp.maximum(h, 0.0)
        h_ref[...] = h
        y2_ref[...] = (
            jnp.dot(h, w_ref[...], preferred_element_type=jnp.float32)
            * dinv
        )

    return pl.pallas_call(
        body,
        grid=(npad // br,),
        in_specs=[
            pl.BlockSpec((1, br, d), lambda i: (0, i, 0)),
            pl.BlockSpec((1, br, d), lambda i: (1, i, 0)),
            pl.BlockSpec((br, d), lambda i: (i, 0)),
            pl.BlockSpec((br, 1), lambda i: (i, 0)),
            pl.BlockSpec((1, d), lambda i: (0, 0)),
            pl.BlockSpec((d, d), lambda i: (0, 0)),
        ],
        out_specs=[
            pl.BlockSpec((br, d), lambda i: (i, 0)),
            pl.BlockSpec((br, d), lambda i: (i, 0)),
        ],
        out_shape=[
            jax.ShapeDtypeStruct((npad, d), jnp.float32),
            jax.ShapeDtypeStruct((npad, d), jnp.float32),
        ],
    )(parts, parts, y, deg2d, b2d, w)


def _finish(parts, y, deg2d, b2d, npad, d, br):
    def body(p0_ref, p1_ref, y_ref, deg_ref, b_ref, h_ref):
        dinv = lax.rsqrt(deg_ref[...] + 1.0)
        h = dinv * (p0_ref[0] + p1_ref[0] + y_ref[...]) + b_ref[...]
        h_ref[...] = jnp.maximum(h, 0.0)

    return pl.pallas_call(
        body,
        grid=(npad // br,),
        in_specs=[
            pl.BlockSpec((1, br, d), lambda i: (0, i, 0)),
            pl.BlockSpec((1, br, d), lambda i: (1, i, 0)),
            pl.BlockSpec((br, d), lambda i: (i, 0)),
            pl.BlockSpec((br, 1), lambda i: (i, 0)),
            pl.BlockSpec((1, d), lambda i: (0, 0)),
        ],
        out_specs=pl.BlockSpec((br, d), lambda i: (i, 0)),
        out_shape=jax.ShapeDtypeStruct((npad, d), jnp.float32),
    )(parts, parts, y, deg2d, b2d)


def kernel(x, edge_index, W1, b1, W2, b2):
    n, d = x.shape
    e = edge_index.shape[1]

    npad = ((n + 16 + 255) // 256) * 256
    rt = npad // NS
    ew = ((e + NW * 1024 - 1) // (NW * 1024)) * 1024
    epad = ew * NW
    chw = ew // 128
    ch1 = epad // NS // 128
    br = 1024

    src = edge_index[0].astype(jnp.int32)
    dst = edge_index[1].astype(jnp.int32)
    pad = epad - e
    pad_src = (jnp.arange(pad, dtype=jnp.int32) * 127) % n
    pad_dst = n + (jnp.arange(pad, dtype=jnp.int32) % (npad - n))
    srcp = jnp.concatenate([src, pad_src]).reshape(epad // 128, 128)
    dstp = jnp.concatenate([dst, pad_dst]).reshape(epad // 128, 128)
    x_pad = jnp.concatenate([x, jnp.zeros((npad - n, d), x.dtype)])
    zcol = jnp.zeros((rt,), jnp.float32)
    zrows = jnp.zeros((rt, d), jnp.float32)

    deg_k = _make_deg(npad, rt, ch1)
    agg = _make_agg(npad, d, rt, chw)

    deg = deg_k(dstp, zcol)
    deg2d = deg.reshape(npad, 1)

    b1r = b1.reshape(1, d)
    b2r = b2.reshape(1, d)

    y1 = _mm_scale(x_pad, W1, deg2d, npad, d, br)
    parts1 = agg(y1, srcp, dstp, zrows)
    h0, y2 = _finish_mm(parts1, y1, deg2d, b1r, W2, npad, d, br)
    parts2 = agg(y2, srcp, dstp, zrows)
    h1 = _finish(parts2, y2, deg2d, b2r, npad, d, br)

    return (h0[:n], h1[:n])


# speedup vs baseline: 28.4592x; 1.3139x over previous
import functools

import jax
import jax.numpy as jnp
from jax import lax
from jax.experimental import pallas as pl
from jax.experimental.pallas import tpu as pltpu
from jax.experimental.pallas import tpu_sc as plsc

NC = 2
NS = 16
NW = NC * NS


def _make_mesh():
    return plsc.VectorSubcoreMesh(core_axis_name="c", subcore_axis_name="s")


def _make_deg(npad, rt, ch1):
    @functools.partial(
        pl.kernel,
        mesh=_make_mesh(),
        out_type=jax.ShapeDtypeStruct((npad,), jnp.float32),
        scratch_types=[
            pltpu.VMEM((ch1, 128), jnp.int32),
            pltpu.VMEM((128,), jnp.float32),
            pltpu.VMEM_SHARED((npad,), jnp.float32),
        ],
    )
    def kfn(dstp_hbm, zcol_hbm, deg_hbm, dst_i, ones_v, deg_sh):
        cid = lax.axis_index("c")
        sid = lax.axis_index("s")

        @pl.when(cid == 0)
        def _():
            pltpu.sync_copy(zcol_hbm, deg_sh.at[pl.ds(sid * rt, rt)])

            def fill(k, carry):
                ones_v[pl.ds(k * 16, 16)] = jnp.full((16,), 1.0, jnp.float32)
                return carry

            lax.fori_loop(0, 8, fill, 0)
            pltpu.sync_copy(dstp_hbm.at[pl.ds(sid * ch1, ch1), :], dst_i)
            plsc.subcore_barrier()

            def body(j, carry):
                pltpu.sync_copy(ones_v, deg_sh.at[dst_i.at[j]], add=True)
                return carry

            lax.fori_loop(0, ch1, body, 0)
            plsc.subcore_barrier()
            pltpu.sync_copy(deg_sh.at[pl.ds(sid * rt, rt)],
                            deg_hbm.at[pl.ds(sid * rt, rt)])

    return kfn


def _make_agg(npad, d, rt, chw):
    bi = 16
    nb = chw // bi

    @functools.partial(
        pl.kernel,
        mesh=_make_mesh(),
        out_type=jax.ShapeDtypeStruct((NC, npad, d), jnp.float32),
        scratch_types=[
            pltpu.VMEM((bi, 128), jnp.int32),
            pltpu.VMEM((bi, 128), jnp.int32),
            pltpu.VMEM((128, d), jnp.float32),
            pltpu.VMEM((128, d), jnp.float32),
            pltpu.VMEM_SHARED((npad, d), jnp.float32),
            pltpu.SemaphoreType.DMA,
            pltpu.SemaphoreType.DMA,
        ],
    )
    def kfn(y_hbm, srcp_hbm, dstp_hbm, zrows_hbm, out_hbm, src_b, dst_b, rows0,
            rows1, acc, sem0, sem1):
        cid = lax.axis_index("c")
        sid = lax.axis_index("s")
        wid = cid * NS + sid

        pltpu.sync_copy(zrows_hbm, acc.at[pl.ds(sid * rt, rt), :])
        plsc.subcore_barrier()

        def blk(b, carry):
            base = wid * chw + b * bi
            pltpu.sync_copy(srcp_hbm.at[pl.ds(base, bi), :], src_b)
            pltpu.sync_copy(dstp_hbm.at[pl.ds(base, bi), :], dst_b)
            pltpu.async_copy(y_hbm.at[src_b.at[0]], rows0, sem0)

            def body(t, c):
                j0 = 2 * t
                j1 = j0 + 1
                pltpu.make_async_copy(y_hbm.at[src_b.at[j0]], rows0,
                                      sem0).wait()
                pltpu.async_copy(y_hbm.at[src_b.at[j1]], rows1, sem1)
                pltpu.sync_copy(rows0, acc.at[dst_b.at[j0]], add=True)
                jn = jnp.where(j0 + 2 < bi, j0 + 2, 0)
                pltpu.async_copy(y_hbm.at[src_b.at[jn]], rows0, sem0)
                pltpu.make_async_copy(y_hbm.at[src_b.at[j1]], rows1,
                                      sem1).wait()
                pltpu.sync_copy(rows1, acc.at[dst_b.at[j1]], add=True)
                return c

            lax.fori_loop(0, bi // 2, body, 0)
            pltpu.make_async_copy(y_hbm.at[src_b.at[0]], rows0, sem0).wait()
            return carry

        lax.fori_loop(0, nb, blk, 0)
        plsc.subcore_barrier()
        pltpu.sync_copy(acc.at[pl.ds(sid * rt, rt), :],
                        out_hbm.at[cid, pl.ds(sid * rt, rt), :])

    return kfn


def _mm_scale(x_pad, w, deg2d, npad, d, br):
    def body(x_ref, w_ref, deg_ref, y_ref):
        dinv = lax.rsqrt(deg_ref[...] + 1.0)
        y_ref[...] = (
            jnp.dot(x_ref[...], w_ref[...], preferred_element_type=jnp.float32)
            * dinv
        )

    return pl.pallas_call(
        body,
        grid=(npad // br,),
        in_specs=[
            pl.BlockSpec((br, d), lambda i: (i, 0)),
            pl.BlockSpec((d, d), lambda i: (0, 0)),
            pl.BlockSpec((br, 1), lambda i: (i, 0)),
        ],
        out_specs=pl.BlockSpec((br, d), lambda i: (i, 0)),
        out_shape=jax.ShapeDtypeStruct((npad, d), jnp.float32),
    )(x_pad, w, deg2d)


def _finish_mm(parts, y, deg2d, b2d, w, npad, d, br):
    def body(p0_ref, p1_ref, y_ref, deg_ref, b_ref, w_ref, h_ref, y2_ref):
        dinv = lax.rsqrt(deg_ref[...] + 1.0)
        h = dinv * (p0_ref[0] + p1_ref[0] + y_ref[...]) + b_ref[...]
        h = jnp.maximum(h, 0.0)
        h_ref[...] = h
        y2_ref[...] = (
            jnp.dot(h, w_ref[...], preferred_element_type=jnp.float32)
            * dinv
        )

    return pl.pallas_call(
        body,
        grid=(npad // br,),
        in_specs=[
            pl.BlockSpec((1, br, d), lambda i: (0, i, 0)),
            pl.BlockSpec((1, br, d), lambda i: (1, i, 0)),
            pl.BlockSpec((br, d), lambda i: (i, 0)),
            pl.BlockSpec((br, 1), lambda i: (i, 0)),
            pl.BlockSpec((1, d), lambda i: (0, 0)),
            pl.BlockSpec((d, d), lambda i: (0, 0)),
        ],
        out_specs=[
            pl.BlockSpec((br, d), lambda i: (i, 0)),
            pl.BlockSpec((br, d), lambda i: (i, 0)),
        ],
        out_shape=[
            jax.ShapeDtypeStruct((npad, d), jnp.float32),
            jax.ShapeDtypeStruct((npad, d), jnp.float32),
        ],
    )(parts, parts, y, deg2d, b2d, w)


def _finish(parts, y, deg2d, b2d, npad, d, br):
    def body(p0_ref, p1_ref, y_ref, deg_ref, b_ref, h_ref):
        dinv = lax.rsqrt(deg_ref[...] + 1.0)
        h = dinv * (p0_ref[0] + p1_ref[0] + y_ref[...]) + b_ref[...]
        h_ref[...] = jnp.maximum(h, 0.0)

    return pl.pallas_call(
        body,
        grid=(npad // br,),
        in_specs=[
            pl.BlockSpec((1, br, d), lambda i: (0, i, 0)),
            pl.BlockSpec((1, br, d), lambda i: (1, i, 0)),
            pl.BlockSpec((br, d), lambda i: (i, 0)),
            pl.BlockSpec((br, 1), lambda i: (i, 0)),
            pl.BlockSpec((1, d), lambda i: (0, 0)),
        ],
        out_specs=pl.BlockSpec((br, d), lambda i: (i, 0)),
        out_shape=jax.ShapeDtypeStruct((npad, d), jnp.float32),
    )(parts, parts, y, deg2d, b2d)


def kernel(x, edge_index, W1, b1, W2, b2):
    n, d = x.shape
    e = edge_index.shape[1]

    npad = ((n + 16 + 255) // 256) * 256
    rt = npad // NS
    ew = ((e + NW * 1024 - 1) // (NW * 1024)) * 1024
    epad = ew * NW
    chw = ew // 128
    ch1 = epad // NS // 128
    br = 1024

    src = edge_index[0].astype(jnp.int32)
    dst = edge_index[1].astype(jnp.int32)
    pad = epad - e
    pad_src = (jnp.arange(pad, dtype=jnp.int32) * 127) % n
    pad_dst = n + (jnp.arange(pad, dtype=jnp.int32) % (npad - n))
    srcp = jnp.concatenate([src, pad_src]).reshape(epad // 128, 128)
    dstp = jnp.concatenate([dst, pad_dst]).reshape(epad // 128, 128)
    x_pad = jnp.concatenate([x, jnp.zeros((npad - n, d), x.dtype)])
    zcol = jnp.zeros((rt,), jnp.float32)
    zrows = jnp.zeros((rt, d), jnp.float32)

    deg_k = _make_deg(npad, rt, ch1)
    agg = _make_agg(npad, d, rt, chw)

    deg = deg_k(dstp, zcol)
    deg2d = deg.reshape(npad, 1)

    b1r = b1.reshape(1, d)
    b2r = b2.reshape(1, d)

    y1 = _mm_scale(x_pad, W1, deg2d, npad, d, br)
    parts1 = agg(y1, srcp, dstp, zrows)
    h0, y2 = _finish_mm(parts1, y1, deg2d, b1r, W2, npad, d, br)
    parts2 = agg(y2, srcp, dstp, zrows)
    h1 = _finish(parts2, y2, deg2d, b2r, npad, d, br)

    return (h0[:n], h1[:n])


# speedup vs baseline: 28.8203x; 1.0127x over previous
import functools

import jax
import jax.numpy as jnp
from jax import lax
from jax.experimental import pallas as pl
from jax.experimental.pallas import tpu as pltpu
from jax.experimental.pallas import tpu_sc as plsc

NC = 2
NS = 16
NW = NC * NS


def _make_mesh():
    return plsc.VectorSubcoreMesh(core_axis_name="c", subcore_axis_name="s")


def _make_deg(npad, rt, ch1):
    @functools.partial(
        pl.kernel,
        mesh=_make_mesh(),
        out_type=jax.ShapeDtypeStruct((NC, npad), jnp.float32),
        scratch_types=[
            pltpu.VMEM((ch1, 128), jnp.int32),
            pltpu.VMEM((128,), jnp.float32),
            pltpu.VMEM_SHARED((npad,), jnp.float32),
        ],
    )
    def kfn(dstp_hbm, zcol_hbm, deg_hbm, dst_i, ones_v, deg_sh):
        cid = lax.axis_index("c")
        sid = lax.axis_index("s")
        wid = cid * NS + sid

        pltpu.sync_copy(zcol_hbm, deg_sh.at[pl.ds(sid * rt, rt)])

        def fill(k, carry):
            ones_v[pl.ds(k * 16, 16)] = jnp.full((16,), 1.0, jnp.float32)
            return carry

        lax.fori_loop(0, 8, fill, 0)
        pltpu.sync_copy(dstp_hbm.at[pl.ds(wid * ch1, ch1), :], dst_i)
        plsc.subcore_barrier()

        def body(j, carry):
            pltpu.sync_copy(ones_v, deg_sh.at[dst_i.at[j]], add=True)
            return carry

        lax.fori_loop(0, ch1, body, 0)
        plsc.subcore_barrier()
        pltpu.sync_copy(deg_sh.at[pl.ds(sid * rt, rt)],
                        deg_hbm.at[cid, pl.ds(sid * rt, rt)])

    return kfn


def _make_agg(npad, d, rt, chw):
    bi = 16
    nb = chw // bi

    @functools.partial(
        pl.kernel,
        mesh=_make_mesh(),
        out_type=jax.ShapeDtypeStruct((NC, npad, d), jnp.float32),
        scratch_types=[
            pltpu.VMEM((bi, 128), jnp.int32),
            pltpu.VMEM((bi, 128), jnp.int32),
            pltpu.VMEM((128, d), jnp.float32),
            pltpu.VMEM((128, d), jnp.float32),
            pltpu.VMEM_SHARED((npad, d), jnp.float32),
            pltpu.SemaphoreType.DMA,
            pltpu.SemaphoreType.DMA,
        ],
    )
    def kfn(y_hbm, srcp_hbm, dstp_hbm, zrows_hbm, out_hbm, src_b, dst_b, rows0,
            rows1, acc, sem0, sem1):
        cid = lax.axis_index("c")
        sid = lax.axis_index("s")
        wid = cid * NS + sid

        pltpu.sync_copy(zrows_hbm, acc.at[pl.ds(sid * rt, rt), :])
        plsc.subcore_barrier()

        def blk(b, carry):
            base = wid * chw + b * bi
            pltpu.sync_copy(srcp_hbm.at[pl.ds(base, bi), :], src_b)
            pltpu.sync_copy(dstp_hbm.at[pl.ds(base, bi), :], dst_b)
            pltpu.async_copy(y_hbm.at[src_b.at[0]], rows0, sem0)

            def body(t, c):
                j0 = 2 * t
                j1 = j0 + 1
                pltpu.make_async_copy(y_hbm.at[src_b.at[j0]], rows0,
                                      sem0).wait()
                pltpu.async_copy(y_hbm.at[src_b.at[j1]], rows1, sem1)
                pltpu.sync_copy(rows0, acc.at[dst_b.at[j0]], add=True)

                @pl.when(j0 + 2 < bi)
                def _():
                    pltpu.async_copy(y_hbm.at[src_b.at[j0 + 2]], rows0, sem0)

                pltpu.make_async_copy(y_hbm.at[src_b.at[j1]], rows1,
                                      sem1).wait()
                pltpu.sync_copy(rows1, acc.at[dst_b.at[j1]], add=True)
                return c

            lax.fori_loop(0, bi // 2, body, 0)
            return carry

        lax.fori_loop(0, nb, blk, 0)
        plsc.subcore_barrier()
        pltpu.sync_copy(acc.at[pl.ds(sid * rt, rt), :],
                        out_hbm.at[cid, pl.ds(sid * rt, rt), :])

    return kfn


def _mm_scale(x_pad, w, deg2d, npad, d, br):
    def body(x_ref, w_ref, deg_ref, y_ref):
        dinv = lax.rsqrt(deg_ref[0] + deg_ref[1] + 1.0)
        y_ref[...] = (
            jnp.dot(x_ref[...], w_ref[...], preferred_element_type=jnp.float32)
            * dinv
        )

    return pl.pallas_call(
        body,
        grid=(npad // br,),
        in_specs=[
            pl.BlockSpec((br, d), lambda i: (i, 0)),
            pl.BlockSpec((d, d), lambda i: (0, 0)),
            pl.BlockSpec((2, br, 1), lambda i: (0, i, 0)),
        ],
        out_specs=pl.BlockSpec((br, d), lambda i: (i, 0)),
        out_shape=jax.ShapeDtypeStruct((npad, d), jnp.float32),
    )(x_pad, w, deg2d)


def _finish_mm(parts, y, deg2d, b2d, w, npad, d, br):
    def body(p0_ref, p1_ref, y_ref, deg_ref, b_ref, w_ref, h_ref, y2_ref):
        dinv = lax.rsqrt(deg_ref[0] + deg_ref[1] + 1.0)
        h = dinv * (p0_ref[0] + p1_ref[0] + y_ref[...]) + b_ref[...]
        h = jnp.maximum(h, 0.0)
        h_ref[...] = h
        y2_ref[...] = (
            jnp.dot(h, w_ref[...], preferred_element_type=jnp.float32)
            * dinv
        )

    return pl.pallas_call(
        body,
        grid=(npad // br,),
        in_specs=[
            pl.BlockSpec((1, br, d), lambda i: (0, i, 0)),
            pl.BlockSpec((1, br, d), lambda i: (1, i, 0)),
            pl.BlockSpec((br, d), lambda i: (i, 0)),
            pl.BlockSpec((2, br, 1), lambda i: (0, i, 0)),
            pl.BlockSpec((1, d), lambda i: (0, 0)),
            pl.BlockSpec((d, d), lambda i: (0, 0)),
        ],
        out_specs=[
            pl.BlockSpec((br, d), lambda i: (i, 0)),
            pl.BlockSpec((br, d), lambda i: (i, 0)),
        ],
        out_shape=[
            jax.ShapeDtypeStruct((npad, d), jnp.float32),
            jax.ShapeDtypeStruct((npad, d), jnp.float32),
        ],
    )(parts, parts, y, deg2d, b2d, w)


def _finish(parts, y, deg2d, b2d, npad, d, br):
    def body(p0_ref, p1_ref, y_ref, deg_ref, b_ref, h_ref):
        dinv = lax.rsqrt(deg_ref[0] + deg_ref[1] + 1.0)
        h = dinv * (p0_ref[0] + p1_ref[0] + y_ref[...]) + b_ref[...]
        h_ref[...] = jnp.maximum(h, 0.0)

    return pl.pallas_call(
        body,
        grid=(npad // br,),
        in_specs=[
            pl.BlockSpec((1, br, d), lambda i: (0, i, 0)),
            pl.BlockSpec((1, br, d), lambda i: (1, i, 0)),
            pl.BlockSpec((br, d), lambda i: (i, 0)),
            pl.BlockSpec((2, br, 1), lambda i: (0, i, 0)),
            pl.BlockSpec((1, d), lambda i: (0, 0)),
        ],
        out_specs=pl.BlockSpec((br, d), lambda i: (i, 0)),
        out_shape=jax.ShapeDtypeStruct((npad, d), jnp.float32),
    )(parts, parts, y, deg2d, b2d)


def kernel(x, edge_index, W1, b1, W2, b2):
    n, d = x.shape
    e = edge_index.shape[1]

    npad = ((n + 16 + 255) // 256) * 256
    rt = npad // NS
    ew = ((e + NW * 1024 - 1) // (NW * 1024)) * 1024
    epad = ew * NW
    chw = ew // 128
    ch1 = epad // NW // 128
    br = 1024

    src = edge_index[0].astype(jnp.int32)
    dst = edge_index[1].astype(jnp.int32)
    pad = epad - e
    pad_src = (jnp.arange(pad, dtype=jnp.int32) * 127) % n
    pad_dst = n + (jnp.arange(pad, dtype=jnp.int32) % (npad - n))
    srcp = jnp.concatenate([src, pad_src]).reshape(epad // 128, 128)
    dstp = jnp.concatenate([dst, pad_dst]).reshape(epad // 128, 128)
    x_pad = jnp.concatenate([x, jnp.zeros((npad - n, d), x.dtype)])
    zcol = jnp.zeros((rt,), jnp.float32)
    zrows = jnp.zeros((rt, d), jnp.float32)

    deg_k = _make_deg(npad, rt, ch1)
    agg = _make_agg(npad, d, rt, chw)

    deg = deg_k(dstp, zcol)
    deg2d = deg.reshape(NC, npad, 1)

    b1r = b1.reshape(1, d)
    b2r = b2.reshape(1, d)

    y1 = _mm_scale(x_pad, W1, deg2d, npad, d, br)
    parts1 = agg(y1, srcp, dstp, zrows)
    h0, y2 = _finish_mm(parts1, y1, deg2d, b1r, W2, npad, d, br)
    parts2 = agg(y2, srcp, dstp, zrows)
    h1 = _finish(parts2, y2, deg2d, b2r, npad, d, br)

    return (h0[:n], h1[:n])


# speedup vs baseline: 31.2956x; 1.0859x over previous
import functools

import jax
import jax.numpy as jnp
from jax import lax
from jax.experimental import pallas as pl
from jax.experimental.pallas import tpu as pltpu
from jax.experimental.pallas import tpu_sc as plsc

NC = 2
NS = 16
NW = NC * NS


def _make_mesh():
    return plsc.VectorSubcoreMesh(core_axis_name="c", subcore_axis_name="s")


def _make_deg(npad, rt, ch1):
    @functools.partial(
        pl.kernel,
        mesh=_make_mesh(),
        out_type=jax.ShapeDtypeStruct((NC, npad), jnp.float32),
        scratch_types=[
            pltpu.VMEM((ch1, 128), jnp.int32),
            pltpu.VMEM((128,), jnp.float32),
            pltpu.VMEM((rt,), jnp.float32),
            pltpu.VMEM_SHARED((npad,), jnp.float32),
        ],
    )
    def kfn(dstp_hbm, deg_hbm, dst_i, ones_v, zv, deg_sh):
        cid = lax.axis_index("c")
        sid = lax.axis_index("s")
        wid = cid * NS + sid

        def zstore(k, c):
            zv[pl.ds(k * 16, 16)] = jnp.zeros((16,), jnp.float32)
            return c

        lax.fori_loop(0, rt // 16, zstore, 0)
        pltpu.sync_copy(zv, deg_sh.at[pl.ds(sid * rt, rt)])

        def fill(k, carry):
            ones_v[pl.ds(k * 16, 16)] = jnp.full((16,), 1.0, jnp.float32)
            return carry

        lax.fori_loop(0, 8, fill, 0)
        pltpu.sync_copy(dstp_hbm.at[pl.ds(wid * ch1, ch1), :], dst_i)
        plsc.subcore_barrier()

        def body(j, carry):
            pltpu.sync_copy(ones_v, deg_sh.at[dst_i.at[j]], add=True)
            return carry

        lax.fori_loop(0, ch1, body, 0)
        plsc.subcore_barrier()
        pltpu.sync_copy(deg_sh.at[pl.ds(sid * rt, rt)],
                        deg_hbm.at[cid, pl.ds(sid * rt, rt)])

    return kfn


def _make_agg(npad, d, rt, chw):
    bi = 16
    nb = chw // bi
    assert nb * bi == chw and bi % 2 == 0

    @functools.partial(
        pl.kernel,
        mesh=_make_mesh(),
        out_type=jax.ShapeDtypeStruct((NC, npad, d), jnp.float32),
        scratch_types=[
            pltpu.VMEM((bi, 128), jnp.int32),
            pltpu.VMEM((bi, 128), jnp.int32),
            pltpu.VMEM((bi, 128), jnp.int32),
            pltpu.VMEM((bi, 128), jnp.int32),
            pltpu.VMEM((128, d), jnp.float32),
            pltpu.VMEM((128, d), jnp.float32),
            pltpu.VMEM_SHARED((npad, d), jnp.float32),
            pltpu.SemaphoreType.DMA,
            pltpu.SemaphoreType.DMA,
            pltpu.SemaphoreType.DMA,
        ],
    )
    def kfn(y_hbm, srcp_hbm, dstp_hbm, out_hbm, src_a, dst_a,
            src_c, dst_c, rows0, rows1, acc, sem0, sem1, semi):
        cid = lax.axis_index("c")
        sid = lax.axis_index("s")
        wid = cid * NS + sid

        idx_bufs = [(src_a, dst_a), (src_c, dst_c)]

        cp0s = pltpu.async_copy(srcp_hbm.at[pl.ds(wid * chw, bi), :], src_a,
                                semi)
        cp0d = pltpu.async_copy(dstp_hbm.at[pl.ds(wid * chw, bi), :], dst_a,
                                semi)

        def zstore(k, c):
            rows0[k // 8, pl.ds((k % 8) * 16, 16)] = jnp.zeros(
                (16,), jnp.float32)
            return c

        lax.fori_loop(0, 1024, zstore, 0)
        for q in range(rt // 128):
            pltpu.sync_copy(rows0, acc.at[pl.ds(sid * rt + q * 128, 128), :])

        cp0s.wait()
        cp0d.wait()
        pltpu.async_copy(y_hbm.at[src_a.at[0]], rows0, sem0)
        plsc.subcore_barrier()

        for b in range(nb):
            src_b, dst_b = idx_bufs[b % 2]
            src_n, dst_n = idx_bufs[(b + 1) % 2]
            have_next = b + 1 < nb
            if have_next:
                base_n = wid * chw + (b + 1) * bi
                cp_s = pltpu.async_copy(
                    srcp_hbm.at[pl.ds(base_n, bi), :], src_n, semi)
                cp_d = pltpu.async_copy(
                    dstp_hbm.at[pl.ds(base_n, bi), :], dst_n, semi)

            def body(t, c, src_b=src_b, dst_b=dst_b):
                j0 = 2 * t
                j1 = j0 + 1
                pltpu.async_copy(y_hbm.at[src_b.at[j1]], rows1, sem1)
                pltpu.make_async_copy(y_hbm.at[src_b.at[j0]], rows0,
                                      sem0).wait()
                pltpu.sync_copy(rows0, acc.at[dst_b.at[j0]], add=True)

                @pl.when(j0 + 2 < bi)
                def _():
                    pltpu.async_copy(y_hbm.at[src_b.at[j0 + 2]], rows0, sem0)

                pltpu.make_async_copy(y_hbm.at[src_b.at[j1]], rows1,
                                      sem1).wait()
                pltpu.sync_copy(rows1, acc.at[dst_b.at[j1]], add=True)
                return c

            lax.fori_loop(0, bi // 2, body, 0)
            if have_next:
                cp_s.wait()
                cp_d.wait()
                pltpu.async_copy(y_hbm.at[src_n.at[0]], rows0, sem0)

        plsc.subcore_barrier()
        pltpu.sync_copy(acc.at[pl.ds(sid * rt, rt), :],
                        out_hbm.at[cid, pl.ds(sid * rt, rt), :])

    return kfn


def _mm(x_rows, w, nrows, d, br):
    def body(x_ref, w_ref, y_ref):
        y_ref[...] = jnp.dot(x_ref[...], w_ref[...],
                             preferred_element_type=jnp.float32)

    return pl.pallas_call(
        body,
        grid=(nrows // br,),
        in_specs=[
            pl.BlockSpec((br, d), lambda i: (i, 0)),
            pl.BlockSpec((d, d), lambda i: (0, 0)),
        ],
        out_specs=pl.BlockSpec((br, d), lambda i: (i, 0)),
        out_shape=jax.ShapeDtypeStruct((nrows, d), jnp.float32),
    )(x_rows, w)


def _scale(xw, deg2d, nrows, d, br):
    def body(xw_ref, deg_ref, y_ref):
        dinv = lax.rsqrt(deg_ref[0] + deg_ref[1] + 1.0)
        y_ref[...] = xw_ref[...] * dinv

    return pl.pallas_call(
        body,
        grid=(nrows // br,),
        in_specs=[
            pl.BlockSpec((br, d), lambda i: (i, 0)),
            pl.BlockSpec((2, br, 1), lambda i: (0, i, 0)),
        ],
        out_specs=pl.BlockSpec((br, d), lambda i: (i, 0)),
        out_shape=jax.ShapeDtypeStruct((nrows, d), jnp.float32),
    )(xw, deg2d)


def _finish_mm(parts, y, deg2d, b2d, w, nrows, d, br):
    def body(p0_ref, p1_ref, y_ref, deg_ref, b_ref, w_ref, h_ref, y2_ref):
        dinv = lax.rsqrt(deg_ref[0] + deg_ref[1] + 1.0)
        h = dinv * (p0_ref[0] + p1_ref[0] + y_ref[...]) + b_ref[...]
        h = jnp.maximum(h, 0.0)
        h_ref[...] = h
        y2_ref[...] = (
            jnp.dot(h, w_ref[...], preferred_element_type=jnp.float32)
            * dinv
        )

    return pl.pallas_call(
        body,
        grid=(nrows // br,),
        in_specs=[
            pl.BlockSpec((1, br, d), lambda i: (0, i, 0)),
            pl.BlockSpec((1, br, d), lambda i: (1, i, 0)),
            pl.BlockSpec((br, d), lambda i: (i, 0)),
            pl.BlockSpec((2, br, 1), lambda i: (0, i, 0)),
            pl.BlockSpec((1, d), lambda i: (0, 0)),
            pl.BlockSpec((d, d), lambda i: (0, 0)),
        ],
        out_specs=[
            pl.BlockSpec((br, d), lambda i: (i, 0)),
            pl.BlockSpec((br, d), lambda i: (i, 0)),
        ],
        out_shape=[
            jax.ShapeDtypeStruct((nrows, d), jnp.float32),
            jax.ShapeDtypeStruct((nrows, d), jnp.float32),
        ],
    )(parts, parts, y, deg2d, b2d, w)


def _finish(parts, y, deg2d, b2d, nrows, d, br):
    def body(p0_ref, p1_ref, y_ref, deg_ref, b_ref, h_ref):
        dinv = lax.rsqrt(deg_ref[0] + deg_ref[1] + 1.0)
        h = dinv * (p0_ref[0] + p1_ref[0] + y_ref[...]) + b_ref[...]
        h_ref[...] = jnp.maximum(h, 0.0)

    return pl.pallas_call(
        body,
        grid=(nrows // br,),
        in_specs=[
            pl.BlockSpec((1, br, d), lambda i: (0, i, 0)),
            pl.BlockSpec((1, br, d), lambda i: (1, i, 0)),
            pl.BlockSpec((br, d), lambda i: (i, 0)),
            pl.BlockSpec((2, br, 1), lambda i: (0, i, 0)),
            pl.BlockSpec((1, d), lambda i: (0, 0)),
        ],
        out_specs=pl.BlockSpec((br, d), lambda i: (i, 0)),
        out_shape=jax.ShapeDtypeStruct((nrows, d), jnp.float32),
    )(parts, parts, y, deg2d, b2d)


def kernel(x, edge_index, W1, b1, W2, b2):
    n, d = x.shape
    e = edge_index.shape[1]

    npad = ((n + 16 + 255) // 256) * 256
    rt = npad // NS
    ew = ((e + NW * 1024 - 1) // (NW * 1024)) * 1024
    epad = ew * NW
    chw = ew // 128
    ch1 = epad // NW // 128
    br = 1000

    src = edge_index[0].astype(jnp.int32)
    dst = edge_index[1].astype(jnp.int32)
    pad = epad - e
    pad_src = (jnp.arange(pad, dtype=jnp.int32) * 127) % n
    pad_dst = n + (jnp.arange(pad, dtype=jnp.int32) % (npad - n))
    srcp = jnp.concatenate([src, pad_src]).reshape(epad // 128, 128)
    dstp = jnp.concatenate([dst, pad_dst]).reshape(epad // 128, 128)

    deg_k = _make_deg(npad, rt, ch1)
    agg = _make_agg(npad, d, rt, chw)

    xw1 = _mm(x, W1, n, d, br)
    deg = deg_k(dstp)
    deg2d = deg.reshape(NC, npad, 1)

    b1r = b1.reshape(1, d)
    b2r = b2.reshape(1, d)

    y1 = _scale(xw1, deg2d, n, d, br)
    parts1 = agg(y1, srcp, dstp)
    h0, y2 = _finish_mm(parts1, y1, deg2d, b1r, W2, n, d, br)
    parts2 = agg(y2, srcp, dstp)
    h1 = _finish(parts2, y2, deg2d, b2r, n, d, br)

    return (h0, h1)


# speedup vs baseline: 31.5628x; 1.0085x over previous
import functools

import jax
import jax.numpy as jnp
from jax import lax
from jax.experimental import pallas as pl
from jax.experimental.pallas import tpu as pltpu
from jax.experimental.pallas import tpu_sc as plsc

NC = 2
NS = 16
NW = NC * NS


def _make_mesh():
    return plsc.VectorSubcoreMesh(core_axis_name="c", subcore_axis_name="s")


def _make_deg(npad, rt, ch1):
    @functools.partial(
        pl.kernel,
        mesh=_make_mesh(),
        out_type=jax.ShapeDtypeStruct((NC, npad), jnp.float32),
        scratch_types=[
            pltpu.VMEM((ch1, 128), jnp.int32),
            pltpu.VMEM((128,), jnp.float32),
            pltpu.VMEM((rt,), jnp.float32),
            pltpu.VMEM_SHARED((npad,), jnp.float32),
            pltpu.SemaphoreType.DMA,
        ],
    )
    def kfn(dstp_hbm, deg_hbm, dst_i, ones_v, zv, deg_sh, semd):
        cid = lax.axis_index("c")
        sid = lax.axis_index("s")
        wid = cid * NS + sid

        cp_i = pltpu.async_copy(dstp_hbm.at[pl.ds(wid * ch1, ch1), :], dst_i,
                                semd)

        def zstore(k, c):
            zv[pl.ds(k * 16, 16)] = jnp.zeros((16,), jnp.float32)
            return c

        lax.fori_loop(0, rt // 16, zstore, 0)
        pltpu.sync_copy(zv, deg_sh.at[pl.ds(sid * rt, rt)])

        def fill(k, carry):
            ones_v[pl.ds(k * 16, 16)] = jnp.full((16,), 1.0, jnp.float32)
            return carry

        lax.fori_loop(0, 8, fill, 0)
        cp_i.wait()
        plsc.subcore_barrier()

        def body(j, carry):
            pltpu.async_copy(ones_v, deg_sh.at[dst_i.at[j]], semd, add=True)

            @pl.when(j >= 1)
            def _():
                pltpu.make_async_copy(ones_v, deg_sh.at[dst_i.at[j - 1]],
                                      semd).wait()

            return carry

        lax.fori_loop(0, ch1, body, 0)
        pltpu.make_async_copy(ones_v, deg_sh.at[dst_i.at[ch1 - 1]],
                              semd).wait()
        plsc.subcore_barrier()
        pltpu.sync_copy(deg_sh.at[pl.ds(sid * rt, rt)],
                        deg_hbm.at[cid, pl.ds(sid * rt, rt)])

    return kfn


def _make_agg(npad, d, rt, chw):
    bi = 16
    nb = chw // bi
    assert nb * bi == chw and bi % 2 == 0

    @functools.partial(
        pl.kernel,
        mesh=_make_mesh(),
        out_type=jax.ShapeDtypeStruct((NC, npad, d), jnp.float32),
        scratch_types=[
            pltpu.VMEM((bi, 128), jnp.int32),
            pltpu.VMEM((bi, 128), jnp.int32),
            pltpu.VMEM((bi, 128), jnp.int32),
            pltpu.VMEM((bi, 128), jnp.int32),
            pltpu.VMEM((128, d), jnp.float32),
            pltpu.VMEM((128, d), jnp.float32),
            pltpu.VMEM_SHARED((npad, d), jnp.float32),
            pltpu.SemaphoreType.DMA,
            pltpu.SemaphoreType.DMA,
            pltpu.SemaphoreType.DMA,
        ],
    )
    def kfn(y_hbm, srcp_hbm, dstp_hbm, out_hbm, src_a, dst_a,
            src_c, dst_c, rows0, rows1, acc, sem0, sem1, semi):
        cid = lax.axis_index("c")
        sid = lax.axis_index("s")
        wid = cid * NS + sid

        idx_bufs = [(src_a, dst_a), (src_c, dst_c)]

        cp0s = pltpu.async_copy(srcp_hbm.at[pl.ds(wid * chw, bi), :], src_a,
                                semi)
        cp0d = pltpu.async_copy(dstp_hbm.at[pl.ds(wid * chw, bi), :], dst_a,
                                semi)

        def zstore(k, c):
            rows0[k // 8, pl.ds((k % 8) * 16, 16)] = jnp.zeros(
                (16,), jnp.float32)
            return c

        lax.fori_loop(0, 1024, zstore, 0)
        for q in range(rt // 128):
            pltpu.sync_copy(rows0, acc.at[pl.ds(sid * rt + q * 128, 128), :])

        cp0s.wait()
        cp0d.wait()
        pltpu.async_copy(y_hbm.at[src_a.at[0]], rows0, sem0)
        plsc.subcore_barrier()

        for b in range(nb):
            src_b, dst_b = idx_bufs[b % 2]
            src_n, dst_n = idx_bufs[(b + 1) % 2]
            have_next = b + 1 < nb
            if have_next:
                base_n = wid * chw + (b + 1) * bi
                cp_s = pltpu.async_copy(
                    srcp_hbm.at[pl.ds(base_n, bi), :], src_n, semi)
                cp_d = pltpu.async_copy(
                    dstp_hbm.at[pl.ds(base_n, bi), :], dst_n, semi)

            def body(t, c, src_b=src_b, dst_b=dst_b):
                j0 = 2 * t
                j1 = j0 + 1
                pltpu.async_copy(y_hbm.at[src_b.at[j1]], rows1, sem1)
                pltpu.make_async_copy(y_hbm.at[src_b.at[j0]], rows0,
                                      sem0).wait()
                pltpu.sync_copy(rows0, acc.at[dst_b.at[j0]], add=True)

                @pl.when(j0 + 2 < bi)
                def _():
                    pltpu.async_copy(y_hbm.at[src_b.at[j0 + 2]], rows0, sem0)

                pltpu.make_async_copy(y_hbm.at[src_b.at[j1]], rows1,
                                      sem1).wait()
                pltpu.sync_copy(rows1, acc.at[dst_b.at[j1]], add=True)
                return c

            lax.fori_loop(0, bi // 2, body, 0)
            if have_next:
                cp_s.wait()
                cp_d.wait()
                pltpu.async_copy(y_hbm.at[src_n.at[0]], rows0, sem0)

        plsc.subcore_barrier()
        pltpu.sync_copy(acc.at[pl.ds(sid * rt, rt), :],
                        out_hbm.at[cid, pl.ds(sid * rt, rt), :])

    return kfn


def _mm(x_rows, w, nrows, d, br):
    def body(x_ref, w_ref, y_ref):
        y_ref[...] = jnp.dot(x_ref[...], w_ref[...],
                             preferred_element_type=jnp.float32)

    return pl.pallas_call(
        body,
        grid=(nrows // br,),
        in_specs=[
            pl.BlockSpec((br, d), lambda i: (i, 0)),
            pl.BlockSpec((d, d), lambda i: (0, 0)),
        ],
        out_specs=pl.BlockSpec((br, d), lambda i: (i, 0)),
        out_shape=jax.ShapeDtypeStruct((nrows, d), jnp.float32),
    )(x_rows, w)


def _scale(xw, deg2d, nrows, d, br):
    def body(xw_ref, deg_ref, y_ref):
        dinv = lax.rsqrt(deg_ref[0] + deg_ref[1] + 1.0)
        y_ref[...] = xw_ref[...] * dinv

    return pl.pallas_call(
        body,
        grid=(nrows // br,),
        in_specs=[
            pl.BlockSpec((br, d), lambda i: (i, 0)),
            pl.BlockSpec((2, br, 1), lambda i: (0, i, 0)),
        ],
        out_specs=pl.BlockSpec((br, d), lambda i: (i, 0)),
        out_shape=jax.ShapeDtypeStruct((nrows, d), jnp.float32),
    )(xw, deg2d)


def _finish_mm(parts, y, deg2d, b2d, w, nrows, d, br):
    def body(p0_ref, p1_ref, y_ref, deg_ref, b_ref, w_ref, h_ref, y2_ref):
        dinv = lax.rsqrt(deg_ref[0] + deg_ref[1] + 1.0)
        h = dinv * (p0_ref[0] + p1_ref[0] + y_ref[...]) + b_ref[...]
        h = jnp.maximum(h, 0.0)
        h_ref[...] = h
        y2_ref[...] = (
            jnp.dot(h, w_ref[...], preferred_element_type=jnp.float32)
            * dinv
        )

    return pl.pallas_call(
        body,
        grid=(nrows // br,),
        in_specs=[
            pl.BlockSpec((1, br, d), lambda i: (0, i, 0)),
            pl.BlockSpec((1, br, d), lambda i: (1, i, 0)),
            pl.BlockSpec((br, d), lambda i: (i, 0)),
            pl.BlockSpec((2, br, 1), lambda i: (0, i, 0)),
            pl.BlockSpec((1, d), lambda i: (0, 0)),
            pl.BlockSpec((d, d), lambda i: (0, 0)),
        ],
        out_specs=[
            pl.BlockSpec((br, d), lambda i: (i, 0)),
            pl.BlockSpec((br, d), lambda i: (i, 0)),
        ],
        out_shape=[
            jax.ShapeDtypeStruct((nrows, d), jnp.float32),
            jax.ShapeDtypeStruct((nrows, d), jnp.float32),
        ],
    )(parts, parts, y, deg2d, b2d, w)


def _finish(parts, y, deg2d, b2d, nrows, d, br):
    def body(p0_ref, p1_ref, y_ref, deg_ref, b_ref, h_ref):
        dinv = lax.rsqrt(deg_ref[0] + deg_ref[1] + 1.0)
        h = dinv * (p0_ref[0] + p1_ref[0] + y_ref[...]) + b_ref[...]
        h_ref[...] = jnp.maximum(h, 0.0)

    return pl.pallas_call(
        body,
        grid=(nrows // br,),
        in_specs=[
            pl.BlockSpec((1, br, d), lambda i: (0, i, 0)),
            pl.BlockSpec((1, br, d), lambda i: (1, i, 0)),
            pl.BlockSpec((br, d), lambda i: (i, 0)),
            pl.BlockSpec((2, br, 1), lambda i: (0, i, 0)),
            pl.BlockSpec((1, d), lambda i: (0, 0)),
        ],
        out_specs=pl.BlockSpec((br, d), lambda i: (i, 0)),
        out_shape=jax.ShapeDtypeStruct((nrows, d), jnp.float32),
    )(parts, parts, y, deg2d, b2d)


def kernel(x, edge_index, W1, b1, W2, b2):
    n, d = x.shape
    e = edge_index.shape[1]

    npad = ((n + 16 + 255) // 256) * 256
    rt = npad // NS
    ew = ((e + NW * 1024 - 1) // (NW * 1024)) * 1024
    epad = ew * NW
    chw = ew // 128
    ch1 = epad // NW // 128
    br = 1000

    src = edge_index[0].astype(jnp.int32)
    dst = edge_index[1].astype(jnp.int32)
    pad = epad - e
    pad_src = (jnp.arange(pad, dtype=jnp.int32) * 127) % n
    pad_dst = n + (jnp.arange(pad, dtype=jnp.int32) % (npad - n))
    srcp = jnp.concatenate([src, pad_src]).reshape(epad // 128, 128)
    dstp = jnp.concatenate([dst, pad_dst]).reshape(epad // 128, 128)

    deg_k = _make_deg(npad, rt, ch1)
    agg = _make_agg(npad, d, rt, chw)

    xw1 = _mm(x, W1, n, d, br)
    deg = deg_k(dstp)
    deg2d = deg.reshape(NC, npad, 1)

    b1r = b1.reshape(1, d)
    b2r = b2.reshape(1, d)

    y1 = _scale(xw1, deg2d, n, d, br)
    parts1 = agg(y1, srcp, dstp)
    h0, y2 = _finish_mm(parts1, y1, deg2d, b1r, W2, n, d, br)
    parts2 = agg(y2, srcp, dstp)
    h1 = _finish(parts2, y2, deg2d, b2r, n, d, br)

    return (h0, h1)
